# baseline (speedup 1.0000x reference)
import functools

import jax
import jax.numpy as jnp
from jax import lax
from jax.experimental import pallas as pl
from jax.experimental.pallas import tpu as pltpu
from jax.experimental.pallas import tpu_sc as plsc

NC = 2
NS = 16
L = 16
NW = NC * NS
RPT = 320
NPAD = RPT * NW
KB = 96
CHUNK = 4000


def _scatter_max(hp, src, dst, C):
    n_edges = src.shape[0]
    assert n_edges % CHUNK == 0
    nchunks = n_edges // CHUNK
    mesh = plsc.VectorSubcoreMesh(
        core_axis_name="c", subcore_axis_name="s", num_cores=NC, num_subcores=NS
    )

    def body(hp_ref, src_ref, dst_ref, out_ref, acc, gbuf, srcb, dstb, dch, sch, gsem):
        wid = lax.axis_index("s") * NC + lax.axis_index("c")
        lo = wid * RPT

        zero16f = jnp.zeros((L,), jnp.float32)
        trash16 = jnp.full((L,), RPT, jnp.int32)
        zero16i = jnp.zeros((L,), jnp.int32)
        iota = lax.iota(jnp.int32, L)

        def init_row(r, carry):
            for c in range(C // L):
                acc[r, pl.ds(c * L, L)] = zero16f
            return carry

        lax.fori_loop(0, RPT + 1, init_row, 0)

        def reset_batch():
            for i in range(KB // L):
                dstb[pl.ds(i * L, L)] = trash16
                srcb[pl.ds(i * L, L)] = zero16i

        reset_batch()

        def flush():
            pltpu.async_copy(hp_ref.at[srcb], gbuf, gsem).wait()

            def group(g, carry):
                d = dstb[pl.ds(g * L, L)]
                for j in range(L):
                    dj = jnp.max(jnp.where(iota == j, d, 0))
                    e = g * L + j
                    for c in range(C // L):
                        cs = c * L
                        acc[dj, pl.ds(cs, L)] = jnp.maximum(
                            acc[dj, pl.ds(cs, L)], gbuf[e, pl.ds(cs, L)]
                        )
                return carry

            lax.fori_loop(0, KB // L, group, 0)
            reset_batch()

        def scan_chunk(ci, cnt):
            pltpu.sync_copy(dst_ref.at[pl.ds(ci * CHUNK, CHUNK)], dch)
            pltpu.sync_copy(src_ref.at[pl.ds(ci * CHUNK, CHUNK)], sch)

            def step(i, cnt):
                d = dch[pl.ds(i * L, L)]
                dl = d - lo
                m = (dl >= 0) & (dl < RPT)
                s = sch[pl.ds(i * L, L)]
                plsc.store_compressed(dstb.at[pl.ds(cnt, L)], dl, mask=m)
                plsc.store_compressed(srcb.at[pl.ds(cnt, L)], s, mask=m)
                cnt = cnt + jnp.sum(m.astype(jnp.int32))

                def do_flush():
                    flush()
                    return jnp.int32(0)

                return lax.cond(cnt > KB - L, do_flush, lambda: cnt)

            return lax.fori_loop(0, CHUNK // L, step, cnt)

        lax.fori_loop(0, nchunks, scan_chunk, jnp.int32(0))
        flush()
        pltpu.sync_copy(acc.at[pl.ds(0, RPT)], out_ref.at[pl.ds(lo, RPT)])

    kern = pl.kernel(
        body,
        out_type=jax.ShapeDtypeStruct((NPAD, C), jnp.float32),
        mesh=mesh,
        scratch_types=[
            pltpu.VMEM((RPT + 1, C), jnp.float32),
            pltpu.VMEM((KB, C), jnp.float32),
            pltpu.VMEM((KB,), jnp.int32),
            pltpu.VMEM((KB,), jnp.int32),
            pltpu.VMEM((CHUNK,), jnp.int32),
            pltpu.VMEM((CHUNK,), jnp.int32),
            pltpu.SemaphoreType.DMA,
        ],
        compiler_params=pltpu.CompilerParams(needs_layout_passes=False),
    )
    return kern(hp, src, dst)


def _dot_t(x, w):
    return lax.dot_general(x, w, (((1,), (1,)), ((), ())), preferred_element_type=jnp.float32)


def _pool_body(x_ref, w_ref, b_ref, o_ref):
    o_ref[...] = jnp.maximum(_dot_t(x_ref[...], w_ref[...]) + b_ref[...], 0.0)


def _mid_body(a_ref, wl_ref, bl_ref, wp_ref, bp_ref, o_ref):
    t = jnp.maximum(_dot_t(a_ref[...], wl_ref[...]) + bl_ref[...], 0.0)
    norm = jnp.sqrt(jnp.sum(t * t, axis=1, keepdims=True))
    h = t / jnp.maximum(norm, 1e-12)
    o_ref[...] = jnp.maximum(_dot_t(h, wp_ref[...]) + bp_ref[...], 0.0)


def _final_body(a_ref, w_ref, b_ref, o_ref):
    o_ref[...] = _dot_t(a_ref[...], w_ref[...]) + b_ref[...]


def _tc_call(body, n_in, x, *weights, R):
    M, K = x.shape
    N = weights[-2].shape[0] if n_in > 3 else weights[0].shape[0]
    specs = [pl.BlockSpec((R, K), lambda i: (i, 0))]
    for w in weights:
        shp = w.shape if w.ndim == 2 else (1, w.shape[0])
        specs.append(pl.BlockSpec(shp, lambda i: (0, 0)))
    args = [x] + [w if w.ndim == 2 else w.reshape(1, -1) for w in weights]
    return pl.pallas_call(
        body,
        grid=(M // R,),
        in_specs=specs,
        out_specs=pl.BlockSpec((R, N), lambda i: (i, 0)),
        out_shape=jax.ShapeDtypeStruct((M, N), jnp.float32),
    )(*args)


def kernel(node_feats, edge_index, W0p, b0p, W0, b0, bias0,
           W1p, b1p, W1, b1, bias1, W2p, b2p, W2, b2, bias2):
    src = edge_index[0].astype(jnp.int32)
    dst = edge_index[1].astype(jnp.int32)

    hp0 = _tc_call(_pool_body, 3, node_feats, W0p, b0p, R=1000)
    agg0 = _scatter_max(hp0, src, dst, 128)
    hp1 = _tc_call(_mid_body, 5, agg0, W0, b0 + bias0, W1p, b1p, R=1024)
    agg1 = _scatter_max(hp1, src, dst, 256)
    hp2 = _tc_call(_mid_body, 5, agg1, W1, b1 + bias1, W2p, b2p, R=1024)
    agg2 = _scatter_max(hp2, src, dst, 256)
    w2pad = jnp.zeros((128, 256), jnp.float32).at[:64].set(W2)
    b2pad = jnp.zeros((128,), jnp.float32).at[:64].set(b2 + bias2)
    out = _tc_call(_final_body, 3, agg2, w2pad, b2pad, R=1024)
    return out[:10000, :64]
